# baseline (speedup 1.0000x reference)
import functools

import numpy as np
import jax
import jax.numpy as jnp
from jax import lax
from jax.experimental import pallas as pl
from jax.experimental.pallas import tpu as pltpu
from jax.experimental.pallas import tpu_sc as plsc

_NUM_T = 1000

_beta = np.linspace(0.0001, 0.02, _NUM_T).astype(np.float32)
_ac = np.cumprod((1.0 - _beta).astype(np.float32), dtype=np.float32)
_TABLE_A = np.sqrt(_ac).astype(np.float32)
_TABLE_B = np.sqrt(1.0 - _ac).astype(np.float32)

_B = 128
_C = 3
_H = 256
_W = 256
_BB = 4
_L = 16
_NW = _B // _L


@functools.partial(
    pl.kernel,
    out_type=[
        jax.ShapeDtypeStruct((_B,), jnp.float32),
        jax.ShapeDtypeStruct((_B,), jnp.float32),
    ],
    mesh=plsc.VectorSubcoreMesh(core_axis_name="c", subcore_axis_name="s"),
    scratch_types=[
        pltpu.VMEM((_L,), jnp.int32),
        pltpu.VMEM((_L,), jnp.float32),
        pltpu.VMEM((_L,), jnp.float32),
        pltpu.SemaphoreType.DMA,
    ],
)
def _sc_gather(t_hbm, ta_hbm, tb_hbm, oa_hbm, ob_hbm, t_v, a_v, b_v, sem):
    nc = 2
    wid = lax.axis_index("s") * nc + lax.axis_index("c")

    @pl.when(wid < _NW)
    def _():
        base = wid * _L
        pltpu.sync_copy(t_hbm.at[pl.ds(base, _L)], t_v)
        pltpu.async_copy(ta_hbm.at[t_v], a_v, sem).wait()
        pltpu.async_copy(tb_hbm.at[t_v], b_v, sem).wait()
        pltpu.sync_copy(a_v, oa_hbm.at[pl.ds(base, _L)])
        pltpu.sync_copy(b_v, ob_hbm.at[pl.ds(base, _L)])


def _tc_body(ca_ref, cb_ref, x_ref, n_ref, o_ref):
    g = pl.program_id(0)
    for i in range(_BB):
        a = ca_ref[g * _BB + i]
        c = cb_ref[g * _BB + i]
        o_ref[i] = a * x_ref[i] + c * n_ref[i]


def kernel(x_start, t, noise):
    ti = t.astype(jnp.int32)
    ta = jnp.asarray(_TABLE_A)
    tb = jnp.asarray(_TABLE_B)

    ca, cb = _sc_gather(ti, ta, tb)

    blk = (_BB, _C, _H, _W)
    grid_spec = pltpu.PrefetchScalarGridSpec(
        num_scalar_prefetch=2,
        grid=(_B // _BB,),
        in_specs=[
            pl.BlockSpec(blk, lambda b, *_: (b, 0, 0, 0)),
            pl.BlockSpec(blk, lambda b, *_: (b, 0, 0, 0)),
        ],
        out_specs=pl.BlockSpec(blk, lambda b, *_: (b, 0, 0, 0)),
    )
    out = pl.pallas_call(
        _tc_body,
        grid_spec=grid_spec,
        out_shape=jax.ShapeDtypeStruct((_B, _C, _H, _W), jnp.float32),
        compiler_params=pltpu.CompilerParams(
            dimension_semantics=("parallel",),
        ),
    )(ca, cb, x_start, noise)
    return out


# speedup vs baseline: 1.2128x; 1.2128x over previous
import numpy as np
import jax
import jax.numpy as jnp
from jax.experimental import pallas as pl
from jax.experimental.pallas import tpu as pltpu

_NUM_T = 1000

_beta = np.linspace(0.0001, 0.02, _NUM_T).astype(np.float32)
_ac = np.cumprod((1.0 - _beta).astype(np.float32), dtype=np.float32)
_TABLE_A = np.sqrt(_ac).astype(np.float32)
_TABLE_B = np.sqrt(1.0 - _ac).astype(np.float32)

_B = 128
_C = 3
_H = 256
_W = 256


_BB = 4


def _body(t_ref, ta_ref, tb_ref, x_ref, n_ref, o_ref):
    g = pl.program_id(0)
    for i in range(_BB):
        ti = t_ref[g * _BB + i]
        a = ta_ref[ti]
        c = tb_ref[ti]
        o_ref[i] = a * x_ref[i] + c * n_ref[i]


def kernel(x_start, t, noise):
    ti = t.astype(jnp.int32)
    ta = jnp.asarray(_TABLE_A)
    tb = jnp.asarray(_TABLE_B)

    blk = (_BB, _C, _H, _W)
    grid_spec = pltpu.PrefetchScalarGridSpec(
        num_scalar_prefetch=3,
        grid=(_B // _BB,),
        in_specs=[
            pl.BlockSpec(blk, lambda b, *_: (b, 0, 0, 0)),
            pl.BlockSpec(blk, lambda b, *_: (b, 0, 0, 0)),
        ],
        out_specs=pl.BlockSpec(blk, lambda b, *_: (b, 0, 0, 0)),
    )
    out = pl.pallas_call(
        _body,
        grid_spec=grid_spec,
        out_shape=jax.ShapeDtypeStruct((_B, _C, _H, _W), jnp.float32),
        compiler_params=pltpu.CompilerParams(
            dimension_semantics=("parallel",),
        ),
    )(ti, ta, tb, x_start, noise)
    return out
